# baseline (speedup 1.0000x reference)
import jax
import jax.numpy as jnp
from jax import lax
from jax.experimental import pallas as pl
from jax.experimental.pallas import tpu as pltpu


def _rup(n, m):
    return ((n + m - 1) // m) * m


def _rot_lanes(v, s, n):
    s = s % n
    if s == 0:
        return v
    return jnp.concatenate([v[:, s:], v[:, :s]], axis=1)


@jax.jit
def _supcon_fwd(x, w_conv, b_conv, w_fc, b_fc):
    B, C, H, W = x.shape
    Cout = w_conv.shape[0]
    N = w_fc.shape[0]
    kh, kw = w_conv.shape[2], w_conv.shape[3]

    P = H * W
    K = kh * kw * C
    Np = _rup(N, 128)
    Wout = 2 * Cout + Np

    TB = 8
    G = B // TB

    xflat = x.reshape(B, C, P)

    wk = jnp.transpose(w_conv, (0, 2, 3, 1)).reshape(Cout, K)
    wk_aug = jnp.concatenate(
        [wk * (1.0 / P), b_conv.reshape(Cout, 1) * (1.0 / P)], axis=1
    ).astype(jnp.bfloat16)

    ii, jj = jnp.meshgrid(jnp.arange(H), jnp.arange(W), indexing="ij")
    rows = []
    for di in range(kh):
        for dj in range(kw):
            a, b = di - kh // 2, dj - kw // 2
            ok = ((ii + a >= 0) & (ii + a < H) & (jj + b >= 0) & (jj + b < W))
            rows.append(ok.reshape(1, P))
    masks = jnp.concatenate(rows, axis=0).astype(jnp.float32)

    wfc = (jnp.zeros((Cout, Np), jnp.float32).at[:, :N].set(w_fc.T)
           .astype(jnp.bfloat16))
    bfc = jnp.pad(b_fc, (0, Np - N)).reshape(1, Np).astype(jnp.float32)

    shifts = [(di - kh // 2) * W + (dj - kw // 2)
              for di in range(kh) for dj in range(kw)]

    def body(x_ref, wk_ref, masks_ref, wfc_ref, bfc_ref, out_ref):
        wk_v = wk_ref[...]
        ones_row = jnp.ones((1, P), jnp.float32)
        feats = []
        for t in range(TB):
            img = x_ref[t]
            taps = [
                _rot_lanes(img, s, P) * masks_ref[k:k + 1, :]
                for k, s in enumerate(shifts)
            ]
            slab = jnp.concatenate(taps + [ones_row], axis=0)
            conv = jnp.dot(wk_v, slab.astype(jnp.bfloat16),
                           preferred_element_type=jnp.float32)
            conv = jnp.maximum(conv, 0.0)
            feats.append(jnp.sum(conv, axis=1, keepdims=True))
        feat = jnp.concatenate(feats, axis=1)

        ssq = jnp.sum(feat * feat, axis=0, keepdims=True)
        proj = feat * lax.rsqrt(jnp.maximum(ssq, 1e-24))

        proj_t = proj.T
        feat_t = feat.T
        logits = (jnp.dot(proj_t.astype(jnp.bfloat16), wfc_ref[...],
                          preferred_element_type=jnp.float32) + bfc_ref[...])

        out_ref[...] = jnp.concatenate([proj_t, feat_t, logits], axis=1)

    out = pl.pallas_call(
        body,
        out_shape=jax.ShapeDtypeStruct((B, Wout), jnp.float32),
        grid=(G,),
        in_specs=[
            pl.BlockSpec((TB, C, P), lambda b: (b, 0, 0)),
            pl.BlockSpec((Cout, K + 1), lambda b: (0, 0)),
            pl.BlockSpec((kh * kw, P), lambda b: (0, 0)),
            pl.BlockSpec((Cout, Np), lambda b: (0, 0)),
            pl.BlockSpec((1, Np), lambda b: (0, 0)),
        ],
        out_specs=pl.BlockSpec((TB, Wout), lambda b: (b, 0)),
        compiler_params=pltpu.CompilerParams(
            dimension_semantics=("parallel",),
            vmem_limit_bytes=64 * 1024 * 1024,
        ),
    )(xflat, wk_aug, masks, wfc, bfc)

    proj = out[:, :Cout]
    feat = out[:, Cout:2 * Cout]
    logits = out[:, 2 * Cout:2 * Cout + N]
    return proj, feat, logits


def kernel(x, w_conv, b_conv, w_fc, b_fc):
    return _supcon_fwd(x, w_conv, b_conv, w_fc, b_fc)


# speedup vs baseline: 1.4230x; 1.4230x over previous
import jax
import jax.numpy as jnp
from jax import lax
from jax.experimental import pallas as pl
from jax.experimental.pallas import tpu as pltpu


def _rup(n, m):
    return ((n + m - 1) // m) * m


@jax.jit
def _supcon_fwd(x, w_conv, b_conv, w_fc, b_fc):
    B, C, H, W = x.shape
    Cout = w_conv.shape[0]
    N = w_fc.shape[0]
    kh, kw = w_conv.shape[2], w_conv.shape[3]

    Hp, Wp = H + kh - 1, W + kw - 1
    P = H * W
    L = (H - 1) * Wp + W
    K = kh * kw * C
    Np = _rup(N, 128)
    Wout = 2 * Cout + Np

    TB = 8
    G = B // TB

    xp = jnp.pad(x, ((0, 0), (0, 0), (1, 1), (1, 1))).astype(jnp.bfloat16)
    xflat = xp.reshape(B, C, Hp * Wp)

    wk = jnp.transpose(w_conv, (0, 2, 3, 1)).reshape(Cout, K)
    wk_aug = jnp.concatenate(
        [wk * (1.0 / P),
         b_conv.reshape(Cout, 1) * (1.0 / P),
         jnp.full((Cout, 1), -16.0)], axis=1
    ).astype(jnp.bfloat16)

    pos = jnp.arange(L)
    valid = ((pos % Wp) < W).astype(jnp.float32).reshape(1, L)
    aug = jnp.concatenate([valid, 1.0 - valid], axis=0).astype(jnp.bfloat16)

    wfc = (jnp.zeros((Cout, Np), jnp.float32).at[:, :N].set(w_fc.T)
           .astype(jnp.bfloat16))
    bfc = jnp.pad(b_fc, (0, Np - N)).reshape(1, Np).astype(jnp.float32)

    offs = [di * Wp + dj for di in range(kh) for dj in range(kw)]

    def body(x_ref, wk_ref, aug_ref, wfc_ref, bfc_ref, out_ref):
        wk_v = wk_ref[...]
        aug_v = aug_ref[...]
        feats = []
        for t in range(TB):
            img = x_ref[t]
            slab = jnp.concatenate(
                [img[:, o:o + L] for o in offs] + [aug_v], axis=0)
            conv = jnp.dot(wk_v, slab,
                           preferred_element_type=jnp.float32)
            conv = jnp.maximum(conv, 0.0)
            feats.append(jnp.sum(conv, axis=1, keepdims=True))
        feat = jnp.concatenate(feats, axis=1)

        ssq = jnp.sum(feat * feat, axis=0, keepdims=True)
        proj = feat * lax.rsqrt(jnp.maximum(ssq, 1e-24))

        proj_t = proj.T
        feat_t = feat.T
        logits = (jnp.dot(proj_t.astype(jnp.bfloat16), wfc_ref[...],
                          preferred_element_type=jnp.float32) + bfc_ref[...])

        out_ref[...] = jnp.concatenate([proj_t, feat_t, logits], axis=1)

    out = pl.pallas_call(
        body,
        out_shape=jax.ShapeDtypeStruct((B, Wout), jnp.float32),
        grid=(G,),
        in_specs=[
            pl.BlockSpec((TB, C, Hp * Wp), lambda b: (b, 0, 0)),
            pl.BlockSpec((Cout, K + 2), lambda b: (0, 0)),
            pl.BlockSpec((2, L), lambda b: (0, 0)),
            pl.BlockSpec((Cout, Np), lambda b: (0, 0)),
            pl.BlockSpec((1, Np), lambda b: (0, 0)),
        ],
        out_specs=pl.BlockSpec((TB, Wout), lambda b: (b, 0)),
        compiler_params=pltpu.CompilerParams(
            dimension_semantics=("parallel",),
            vmem_limit_bytes=64 * 1024 * 1024,
        ),
    )(xflat, wk_aug, aug, wfc, bfc)

    proj = out[:, :Cout]
    feat = out[:, Cout:2 * Cout]
    logits = out[:, 2 * Cout:2 * Cout + N]
    return proj, feat, logits


def kernel(x, w_conv, b_conv, w_fc, b_fc):
    return _supcon_fwd(x, w_conv, b_conv, w_fc, b_fc)


# speedup vs baseline: 1.5277x; 1.0736x over previous
import jax
import jax.numpy as jnp
from jax import lax
from jax.experimental import pallas as pl
from jax.experimental.pallas import tpu as pltpu


def _rup(n, m):
    return ((n + m - 1) // m) * m


@jax.jit
def _supcon_fwd(x, w_conv, b_conv, w_fc, b_fc):
    B, C, H, W = x.shape
    Cout = w_conv.shape[0]
    N = w_fc.shape[0]
    kh, kw = w_conv.shape[2], w_conv.shape[3]

    Hp, Wp = H + kh - 1, W + kw - 1
    P = H * W
    L = (H - 1) * Wp + W
    K = kh * kw * C
    Np = _rup(N, 128)
    Wout = 2 * Cout + Np

    TB = 16
    G = B // TB

    xp = jnp.pad(x, ((0, 0), (0, 0), (1, 1), (1, 1))).astype(jnp.bfloat16)
    xflat = xp.reshape(B, C, Hp * Wp)

    wk = jnp.transpose(w_conv, (0, 2, 3, 1)).reshape(Cout, K)
    wk_aug = jnp.concatenate(
        [wk * (1.0 / P),
         b_conv.reshape(Cout, 1) * (1.0 / P),
         jnp.full((Cout, 1), -16.0)], axis=1
    ).astype(jnp.bfloat16)

    pos = jnp.arange(L)
    valid = ((pos % Wp) < W).astype(jnp.float32).reshape(1, L)
    aug = jnp.concatenate([valid, 1.0 - valid], axis=0).astype(jnp.bfloat16)

    wfc = (jnp.zeros((Cout, Np), jnp.float32).at[:, :N].set(w_fc.T)
           .astype(jnp.bfloat16))
    bfc = jnp.pad(b_fc, (0, Np - N)).reshape(1, Np).astype(jnp.float32)

    offs = [di * Wp + dj for di in range(kh) for dj in range(kw)]

    def body(x_ref, wk_ref, aug_ref, wfc_ref, bfc_ref, out_ref):
        wk_v = wk_ref[...]
        aug_v = aug_ref[...]
        feats = []
        for t in range(TB):
            img = x_ref[t]
            slab = jnp.concatenate(
                [img[:, o:o + L] for o in offs] + [aug_v], axis=0)
            conv = jnp.dot(wk_v, slab,
                           preferred_element_type=jnp.float32)
            conv = jnp.maximum(conv, 0.0)
            feats.append(jnp.sum(conv, axis=1, keepdims=True))
        feat = jnp.concatenate(feats, axis=1)

        ssq = jnp.sum(feat * feat, axis=0, keepdims=True)
        proj = feat * lax.rsqrt(jnp.maximum(ssq, 1e-24))

        proj_t = proj.T
        feat_t = feat.T
        logits = (jnp.dot(proj_t.astype(jnp.bfloat16), wfc_ref[...],
                          preferred_element_type=jnp.float32) + bfc_ref[...])

        out_ref[...] = jnp.concatenate([proj_t, feat_t, logits], axis=1)

    out = pl.pallas_call(
        body,
        out_shape=jax.ShapeDtypeStruct((B, Wout), jnp.float32),
        grid=(G,),
        in_specs=[
            pl.BlockSpec((TB, C, Hp * Wp), lambda b: (b, 0, 0)),
            pl.BlockSpec((Cout, K + 2), lambda b: (0, 0)),
            pl.BlockSpec((2, L), lambda b: (0, 0)),
            pl.BlockSpec((Cout, Np), lambda b: (0, 0)),
            pl.BlockSpec((1, Np), lambda b: (0, 0)),
        ],
        out_specs=pl.BlockSpec((TB, Wout), lambda b: (b, 0)),
        compiler_params=pltpu.CompilerParams(
            dimension_semantics=("parallel",),
            vmem_limit_bytes=64 * 1024 * 1024,
        ),
    )(xflat, wk_aug, aug, wfc, bfc)

    proj = out[:, :Cout]
    feat = out[:, Cout:2 * Cout]
    logits = out[:, 2 * Cout:2 * Cout + N]
    return proj, feat, logits


def kernel(x, w_conv, b_conv, w_fc, b_fc):
    return _supcon_fwd(x, w_conv, b_conv, w_fc, b_fc)
